# Initial kernel scaffold; baseline (speedup 1.0000x reference)
import jax
import jax.numpy as jnp
from jax.experimental import pallas as pl


def kernel(x, length, W1, b1, g1, be1, W2, b2, g2, be2):
    raise NotImplementedError("write your pallas kernel here")


# speedup vs baseline: 3.8507x; 3.8507x over previous
import jax
import jax.numpy as jnp
from jax.experimental import pallas as pl

N = 32768
B = 16
D = 32
H = 16
EPS = 1e-5


def _body(x_ref, w1t_ref, b1_ref, g1_ref, be1_ref, w2_ref, b2_ref, g2_ref,
          be2_ref, starts_ref, ends_ref, lenf_ref, out_ref):
    x = x_ref[...]
    h = jnp.dot(x, w1t_ref[...], preferred_element_type=jnp.float32)
    h = h + b1_ref[...]
    m1 = jnp.mean(h, axis=0, keepdims=True)
    v1 = jnp.mean((h - m1) * (h - m1), axis=0, keepdims=True)
    hn = (h - m1) * jax.lax.rsqrt(v1 + EPS) * g1_ref[...] + be1_ref[...]
    hn = jnp.maximum(hn, 0.0)
    o = jnp.sum(hn * w2_ref[...], axis=1, keepdims=True) + b2_ref[0, 0]
    m2 = jnp.mean(o)
    v2 = jnp.mean((o - m2) * (o - m2))
    out1 = (o - m2) * jax.lax.rsqrt(v2 + EPS) * g2_ref[0, 0] + be2_ref[0, 0]
    prod = x * out1
    pos = jax.lax.broadcasted_iota(jnp.int32, (N, B), 0)
    onehot = jnp.where(
        (pos >= starts_ref[...]) & (pos < ends_ref[...]), 1.0, 0.0)
    sums = jax.lax.dot_general(
        onehot, prod, (((0,), (0,)), ((), ())),
        preferred_element_type=jnp.float32)
    means = sums * (1.0 / lenf_ref[...]).reshape(B, 1)
    nrm = jnp.sqrt(jnp.sum(means * means, axis=1, keepdims=True))
    out_ref[...] = means / jnp.maximum(nrm, 1e-12)


def kernel(x, length, W1, b1, g1, be1, W2, b2, g2, be2):
    length = length.astype(jnp.int32)
    ends = jnp.cumsum(length)
    starts = ends - length
    return pl.pallas_call(
        _body,
        out_shape=jax.ShapeDtypeStruct((B, D), jnp.float32),
    )(
        x,
        W1.T,
        b1.reshape(1, H),
        g1.reshape(1, H),
        be1.reshape(1, H),
        W2.reshape(1, H),
        b2.reshape(1, 1),
        g2.reshape(1, 1),
        be2.reshape(1, 1),
        starts.reshape(1, B),
        ends.reshape(1, B),
        length.astype(jnp.float32).reshape(1, B),
    )


# speedup vs baseline: 4.6849x; 1.2166x over previous
import jax
import jax.numpy as jnp
from jax.experimental import pallas as pl

N = 32768
B = 16
D = 32
H = 16
G = 4
R = N // G
EPS = 1e-5


def _body(xp_ref, w1_ref, b1_ref, g1_ref, be1_ref, w2rep_ref, w2blk_ref,
          b2v_ref, g2_ref, be2_ref, thr_ref, lenf_ref, out_ref):
    f32 = jnp.float32
    xp = xp_ref[...]
    hp = jnp.dot(xp, w1_ref[...], preferred_element_type=f32) + b1_ref[...]

    s1 = jnp.sum(hp, axis=0, keepdims=True)
    gram = jax.lax.dot_general(hp, hp, (((0,), (0,)), ((), ())),
                               preferred_element_type=f32)
    i0 = jax.lax.broadcasted_iota(jnp.int32, (G * H, G * H), 0)
    i1 = jax.lax.broadcasted_iota(jnp.int32, (G * H, G * H), 1)
    eye64 = jnp.where(i0 == i1, 1.0, 0.0)
    q1 = jnp.sum(gram * eye64, axis=0, keepdims=True)
    t16 = jnp.where(i0 % H == i1 % H, 1.0, 0.0)
    sq = jnp.concatenate([s1, q1], axis=0)
    sq_rep = jnp.dot(sq, t16, preferred_element_type=f32)
    m1 = sq_rep[0:1, :] * (1.0 / N)
    v1 = sq_rep[1:2, :] * (1.0 / N) - m1 * m1
    sc = g1_ref[...] * jax.lax.rsqrt(v1 + EPS)
    sh = be1_ref[...] - m1 * sc
    hn = jnp.maximum(hp * sc + sh, 0.0)

    op_rep = jnp.dot(hn, w2rep_ref[...], preferred_element_type=f32) \
        + b2v_ref[0, 0]
    op5 = jnp.dot(hn, w2blk_ref[...], preferred_element_type=f32) \
        + b2v_ref[...]
    g5 = jax.lax.dot_general(op5, op5, (((0,), (0,)), ((), ())),
                             preferred_element_type=f32)
    j0 = jax.lax.broadcasted_iota(jnp.int32, (G + 1, G + 1), 0)
    j1 = jax.lax.broadcasted_iota(jnp.int32, (G + 1, G + 1), 1)
    d5 = jnp.where((j0 == j1) & (j0 < G), 1.0, 0.0)
    m5 = jnp.where((j0 == G) & (j1 < G), 1.0, 0.0)
    sum_o2 = jnp.sum(g5 * d5)
    sum_o = jnp.sum(g5 * m5)
    m2 = sum_o * (1.0 / N)
    v2 = sum_o2 * (1.0 / N) - m2 * m2
    a = g2_ref[0, 0] * jax.lax.rsqrt(v2 + EPS)
    c = be2_ref[0, 0] - a * m2

    row = jax.lax.broadcasted_iota(jnp.int32, (R, G * H), 0)
    maskf = jnp.where(row < thr_ref[...], 1.0, 0.0)
    gm = maskf * op_rep
    mm1 = jax.lax.dot_general(gm, xp, (((0,), (0,)), ((), ())),
                              preferred_element_type=f32)
    mm0 = jax.lax.dot_general(maskf, xp, (((0,), (0,)), ((), ())),
                              preferred_element_type=f32)
    p1 = (mm1[0:16, 0:32] + mm1[16:32, 32:64]
          + mm1[32:48, 64:96] + mm1[48:64, 96:128])
    p0 = (mm0[0:16, 0:32] + mm0[16:32, 32:64]
          + mm0[32:48, 64:96] + mm0[48:64, 96:128])
    p = a * p1 + c * p0

    k0 = jax.lax.broadcasted_iota(jnp.int32, (B, B), 0)
    k1 = jax.lax.broadcasted_iota(jnp.int32, (B, B), 1)
    dm = jnp.where(k0 == k1, 1.0, 0.0) - jnp.where(k1 == k0 - 1, 1.0, 0.0)
    s = jnp.dot(dm, p, preferred_element_type=f32)
    means = s / lenf_ref[...]
    nrm = jnp.sqrt(jnp.sum(means * means, axis=1, keepdims=True))
    out_ref[...] = means / jnp.maximum(nrm, 1e-12)


def kernel(x, length, W1, b1, g1, be1, W2, b2, g2, be2):
    f32 = jnp.float32
    length = length.astype(jnp.int32)
    ends = jnp.cumsum(length)
    lane = jnp.arange(G * H, dtype=jnp.int32)
    b_of = lane % H
    j_of = lane // H
    thr = ((ends[b_of] - j_of + 3) // G).reshape(1, G * H)

    eye4 = jnp.eye(G, dtype=f32)
    w1rep = jnp.kron(eye4, W1.T)
    w2col = W2.reshape(H, 1)
    w2rep = jnp.kron(eye4, w2col @ jnp.ones((1, H), f32))
    w2blk = jnp.concatenate(
        [jnp.kron(eye4, w2col), jnp.zeros((G * H, 1), f32)], axis=1)
    b2v = jnp.concatenate(
        [jnp.broadcast_to(b2.astype(f32), (G,)), jnp.ones((1,), f32)]
    ).reshape(1, G + 1)

    return pl.pallas_call(
        _body,
        out_shape=jax.ShapeDtypeStruct((B, D), f32),
    )(
        x.reshape(R, G * D),
        w1rep,
        jnp.tile(b1, G).reshape(1, G * H),
        jnp.tile(g1, G).reshape(1, G * H),
        jnp.tile(be1, G).reshape(1, G * H),
        w2rep,
        w2blk,
        b2v,
        g2.reshape(1, 1),
        be2.reshape(1, 1),
        thr,
        length.astype(f32).reshape(B, 1),
    )
